# baseline (speedup 1.0000x reference)
import jax
import jax.numpy as jnp
from jax import lax
from jax.experimental import pallas as pl
from jax.experimental.pallas import tpu as pltpu
from jax.experimental.pallas import tpu_sc as plsc

_N = 100000
_E = 1600000
_INDIM = 12
_OUTDIM = 32
_HEADS = 6
_EDGE_DIM_ = 4
_HP = 16
_HC = 96
_BE = 128
_NBLK = _E // _BE
_NC, _NS = 2, 16
_NW = _NC * _NS
_NPAD = 100096
_RPS = _NPAD // _NS
_BN = 2000
_BEB = 8000
_QA, _RA = _NBLK // _NW, _NBLK % _NW
_BEB2 = 64
_NBLKB = _E // _BEB2
_QB, _RB = _NBLKB // _NS, _NBLKB % _NS


def _group_mat():
    r = (lax.broadcasted_iota(jnp.int32, (_HEADS * _OUTDIM, _HP), 0) % _HC) // _HP
    c = lax.broadcasted_iota(jnp.int32, (_HEADS * _OUTDIM, _HP), 1)
    return (r == c).astype(jnp.float32)


def _tc_edge_body(ea_ref, bb_ref, pe_ref):
    pe_ref[...] = jnp.dot(ea_ref[...], bb_ref[...],
                          preferred_element_type=jnp.float32)


def _tc_node_body(x_ref, w_ref, asf_ref, adf_ref, h_ref, asrc_ref, adst_ref):
    h = jnp.dot(x_ref[...], w_ref[...], preferred_element_type=jnp.float32)
    h_ref[...] = h
    gm = _group_mat()
    asrc_ref[...] = jnp.dot(h * asf_ref[...], gm,
                            preferred_element_type=jnp.float32)
    adst_ref[...] = jnp.dot(h * adf_ref[...], gm,
                            preferred_element_type=jnp.float32)


def _sc_a_body(srcA, dstA, pe3, asrc, adst, zeros,
               p3, sd3, dpart,
               src0, src1, dst0, dst1,
               pe0, pe1, as0, as1, ad0, ad1, pv0, pv1,
               sl0, sl1, sg0, sg1, sw0, sw1, den_sh):
    c = lax.axis_index("c")
    s = lax.axis_index("s")
    wid = s * _NC + c
    r0 = s * _RPS
    pltpu.sync_copy(zeros.at[pl.ds(r0, _RPS)], den_sh.at[pl.ds(r0, _RPS)])
    plsc.subcore_barrier()

    base = wid * _QA + jnp.minimum(wid, _RA)
    nb = _QA + (wid < _RA).astype(jnp.int32)
    srcv = (src0, src1)
    dstv = (dst0, dst1)
    pev = (pe0, pe1)
    asv = (as0, as1)
    adv = (ad0, ad1)
    pv = (pv0, pv1)
    sl = (sl0, sl1)
    sg = (sg0, sg1)
    sw = (sw0, sw1)

    def fire_lin(b, q):
        pltpu.async_copy(srcA.at[b, 0], srcv[q], sl[q])
        pltpu.async_copy(dstA.at[b, 0], dstv[q], sl[q])
        pltpu.async_copy(pe3.at[b], pev[q], sl[q])

    def substep(i, par):
        b = base + i
        q = 1 - par

        @pl.when(i < nb)
        def _():
            pltpu.make_async_copy(srcA.at[b, 0], srcv[par], sl[par]).wait()
            pltpu.make_async_copy(dstA.at[b, 0], dstv[par], sl[par]).wait()
            pltpu.make_async_copy(pe3.at[b], pev[par], sl[par]).wait()
            pltpu.async_copy(asrc.at[srcv[par]], asv[par], sg[par])
            pltpu.async_copy(adst.at[dstv[par]], adv[par], sg[par])

        @pl.when((i >= 1) & (i - 1 < nb))
        def _():
            pltpu.make_async_copy(asrc.at[srcv[q]], asv[q], sg[q]).wait()
            pltpu.make_async_copy(adst.at[dstv[q]], adv[q], sg[q]).wait()

            @pl.when(i >= 3)
            def _():
                pltpu.make_async_copy(pv[q], p3.at[b], sw[q]).wait()

            a = asv[q][...] + adv[q][...] + pev[q][...]
            a = jnp.where(a >= 0.0, a, 0.2 * a)
            pv[q][...] = jnp.exp(a)
            pltpu.async_copy(pv[q], p3.at[b - 1], sw[q])
            pltpu.async_copy(srcv[q], sd3.at[b - 1, 0], sw[q])
            pltpu.async_copy(dstv[q], sd3.at[b - 1, 1], sw[q])
            pltpu.sync_copy(pv[q], den_sh.at[dstv[q]], add=True)

        @pl.when(i + 1 < nb)
        def _():
            @pl.when(i >= 1)
            def _():
                pltpu.make_async_copy(srcv[q], sd3.at[b, 0], sw[q]).wait()
                pltpu.make_async_copy(dstv[q], sd3.at[b, 1], sw[q]).wait()

            fire_lin(b + 1, q)

    fire_lin(base, 0)

    def pair(i2, carry):
        substep(2 * i2, 0)
        substep(2 * i2 + 1, 1)
        return carry

    lax.fori_loop(0, (nb + 2) // 2, pair, 0)
    for qq in (0, 1):
        pltpu.make_async_copy(pv[qq], p3.at[0], sw[qq]).wait()
        pltpu.make_async_copy(srcv[qq], sd3.at[0, 0], sw[qq]).wait()
        pltpu.make_async_copy(dstv[qq], sd3.at[0, 1], sw[qq]).wait()
    plsc.subcore_barrier()
    pltpu.sync_copy(den_sh.at[pl.ds(r0, _RPS)], dpart.at[c, pl.ds(r0, _RPS)])


def _tc_inv_body(d_ref, o_ref):
    o_ref[...] = 1.0 / (d_ref[0] + d_ref[1] + 1e-16)


def _sc_b_body(sd3, p3, invd, h2, zeros,
               oh,
               src0, src1, dst0, dst1, pb0, pb1, iv0, iv1,
               ix0, ix1, hr0, hr1, w_v, msg_v,
               sl0, sl1, sg0, sg1, acc_sh):
    c = lax.axis_index("c")
    s = lax.axis_index("s")
    r0 = s * _RPS
    pltpu.sync_copy(zeros.at[pl.ds(r0, _RPS)], acc_sh.at[pl.ds(r0, _RPS)])
    plsc.subcore_barrier()

    base = s * _QB + jnp.minimum(s, _RB)
    nb = _QB + (s < _RB).astype(jnp.int32)
    srcv = (src0, src1)
    dstv = (dst0, dst1)
    pbv = (pb0, pb1)
    iv = (iv0, iv1)
    ix = (ix0, ix1)
    hr = (hr0, hr1)
    sl = (sl0, sl1)
    sg = (sg0, sg1)

    def fire_lin(b, q):
        off = (b % 2) * _BEB2
        pltpu.async_copy(sd3.at[b // 2, 0, pl.ds(off, _BEB2)], srcv[q], sl[q])
        pltpu.async_copy(sd3.at[b // 2, 1, pl.ds(off, _BEB2)], dstv[q], sl[q])
        pltpu.async_copy(p3.at[b], pbv[q], sl[q])

    def substep(i, par):
        b = base + i
        q = 1 - par

        @pl.when(i < nb)
        def _():
            off = (b % 2) * _BEB2
            pltpu.make_async_copy(sd3.at[b // 2, 0, pl.ds(off, _BEB2)],
                                  srcv[par], sl[par]).wait()
            pltpu.make_async_copy(sd3.at[b // 2, 1, pl.ds(off, _BEB2)],
                                  dstv[par], sl[par]).wait()
            pltpu.make_async_copy(p3.at[b], pbv[par], sl[par]).wait()
            pltpu.async_copy(invd.at[dstv[par]], iv[par], sg[par])
            ix[par][...] = srcv[par][...] * 2 + c
            pltpu.async_copy(h2.at[ix[par]], hr[par], sg[par])

        @pl.when((i >= 1) & (i - 1 < nb))
        def _():
            pltpu.make_async_copy(invd.at[dstv[q]], iv[q], sg[q]).wait()
            pltpu.make_async_copy(h2.at[ix[q]], hr[q], sg[q]).wait()
            w_v[...] = pbv[q][...] * iv[q][...]
            hq = hr[q]

            def edge(j, carry2):
                w = w_v[j]
                m = w[0] * hq[j, pl.ds(0, _HP)]
                m = m + w[1] * hq[j, pl.ds(16, _HP)]
                m = m + w[2] * hq[j, pl.ds(32, _HP)]
                m = m + w[3] * hq[j, pl.ds(48, _HP)]
                m = m + w[4] * hq[j, pl.ds(64, _HP)]
                m = m + w[5] * hq[j, pl.ds(80, _HP)]
                msg_v[j] = m
                return carry2

            lax.fori_loop(0, _BEB2, edge, 0, unroll=4)
            pltpu.sync_copy(msg_v, acc_sh.at[dstv[q]], add=True)

        @pl.when(i + 1 < nb)
        def _():
            fire_lin(b + 1, q)

    fire_lin(base, 0)

    def pair(i2, carry):
        substep(2 * i2, 0)
        substep(2 * i2 + 1, 1)
        return carry

    lax.fori_loop(0, (nb + 2) // 2, pair, 0)
    plsc.subcore_barrier()
    pltpu.sync_copy(acc_sh.at[pl.ds(r0, _RPS)], oh.at[c, pl.ds(r0, _RPS)])


def _tc_out_body(oh_ref, b0_ref, b1_ref, o0_ref, o1_ref):
    o0_ref[...] = jax.nn.sigmoid(oh_ref[0] * (1.0 / _HEADS) + b0_ref[...])
    o1_ref[...] = jax.nn.sigmoid(oh_ref[1] * (1.0 / _HEADS) + b1_ref[...])


def kernel(x, edge_index, edge_attr, W, lin_edge, att_src, att_dst, att_edge,
           bias):
    f32 = jnp.float32
    srcA = edge_index[0].reshape(_NBLK, 1, _BE)
    dstA = edge_index[1].reshape(_NBLK, 1, _BE)
    Wp = W.reshape(_INDIM, _HEADS, 2, _HP).transpose(0, 2, 1, 3) \
          .reshape(_INDIM, _HEADS * _OUTDIM)
    asf = att_src.reshape(_HEADS, 2, _HP).transpose(1, 0, 2).reshape(1, -1)
    adf = att_dst.reshape(_HEADS, 2, _HP).transpose(1, 0, 2).reshape(1, -1)
    bmat = jnp.einsum('khd,hd->kh', lin_edge.reshape(_EDGE_DIM_, _HEADS,
                                                     _OUTDIM), att_edge)
    bmat = jnp.pad(bmat, ((0, 0), (0, _HP - _HEADS))).astype(f32)
    bigb = jnp.kron(jnp.eye(8, dtype=f32), bmat)
    ea32 = edge_attr.reshape(_E // 8, 8 * _EDGE_DIM_)
    zeros = jnp.zeros((_NPAD, _HP), f32)

    hp, asrc, adst = pl.pallas_call(
        _tc_node_body,
        grid=(_N // _BN,),
        in_specs=[
            pl.BlockSpec((_BN, _INDIM), lambda i: (i, 0)),
            pl.BlockSpec((_INDIM, _HEADS * _OUTDIM), lambda i: (0, 0)),
            pl.BlockSpec((1, _HEADS * _OUTDIM), lambda i: (0, 0)),
            pl.BlockSpec((1, _HEADS * _OUTDIM), lambda i: (0, 0)),
        ],
        out_specs=[
            pl.BlockSpec((_BN, _HEADS * _OUTDIM), lambda i: (i, 0)),
            pl.BlockSpec((_BN, _HP), lambda i: (i, 0)),
            pl.BlockSpec((_BN, _HP), lambda i: (i, 0)),
        ],
        out_shape=[
            jax.ShapeDtypeStruct((_N, _HEADS * _OUTDIM), f32),
            jax.ShapeDtypeStruct((_N, _HP), f32),
            jax.ShapeDtypeStruct((_N, _HP), f32),
        ],
    )(x, Wp, asf, adf)

    pe128 = pl.pallas_call(
        _tc_edge_body,
        grid=(_E // 8 // 1000,),
        in_specs=[
            pl.BlockSpec((1000, 8 * _EDGE_DIM_), lambda i: (i, 0)),
            pl.BlockSpec((8 * _EDGE_DIM_, 128), lambda i: (0, 0)),
        ],
        out_specs=pl.BlockSpec((1000, 128), lambda i: (i, 0)),
        out_shape=jax.ShapeDtypeStruct((_E // 8, 128), f32),
    )(ea32, bigb)
    pe3 = pe128.reshape(_NBLK, _BE, _HP)

    mesh = plsc.VectorSubcoreMesh(core_axis_name="c", subcore_axis_name="s",
                                  num_cores=_NC, num_subcores=_NS)
    p3, sd3, dpart = pl.kernel(
        _sc_a_body,
        out_type=[
            jax.ShapeDtypeStruct((_NBLK, _BE, _HP), f32),
            jax.ShapeDtypeStruct((_NBLK, 2, _BE), jnp.int32),
            jax.ShapeDtypeStruct((_NC, _NPAD, _HP), f32),
        ],
        mesh=mesh,
        compiler_params=pltpu.CompilerParams(use_tc_tiling_on_sc=False),
        scratch_types=[
            pltpu.VMEM((_BE,), jnp.int32), pltpu.VMEM((_BE,), jnp.int32),
            pltpu.VMEM((_BE,), jnp.int32), pltpu.VMEM((_BE,), jnp.int32),
            pltpu.VMEM((_BE, _HP), f32), pltpu.VMEM((_BE, _HP), f32),
            pltpu.VMEM((_BE, _HP), f32), pltpu.VMEM((_BE, _HP), f32),
            pltpu.VMEM((_BE, _HP), f32), pltpu.VMEM((_BE, _HP), f32),
            pltpu.VMEM((_BE, _HP), f32), pltpu.VMEM((_BE, _HP), f32),
            pltpu.SemaphoreType.DMA, pltpu.SemaphoreType.DMA,
            pltpu.SemaphoreType.DMA, pltpu.SemaphoreType.DMA,
            pltpu.SemaphoreType.DMA, pltpu.SemaphoreType.DMA,
            pltpu.VMEM_SHARED((_NPAD, _HP), f32),
        ],
    )(srcA, dstA, pe3, asrc, adst, zeros)

    invd = pl.pallas_call(
        _tc_inv_body,
        grid=(_NPAD // 3128,),
        in_specs=[pl.BlockSpec((_NC, 3128, _HP), lambda i: (0, i, 0))],
        out_specs=pl.BlockSpec((3128, _HP), lambda i: (i, 0)),
        out_shape=jax.ShapeDtypeStruct((_NPAD, _HP), f32),
    )(dpart)

    h2 = hp.reshape(_N * 2, _HC)
    pb3 = p3.reshape(_NBLKB, _BEB2, _HP)
    oh = pl.kernel(
        _sc_b_body,
        out_type=jax.ShapeDtypeStruct((_NC, _NPAD, _HP), f32),
        mesh=mesh,
        compiler_params=pltpu.CompilerParams(use_tc_tiling_on_sc=False),
        scratch_types=[
            pltpu.VMEM((_BEB2,), jnp.int32), pltpu.VMEM((_BEB2,), jnp.int32),
            pltpu.VMEM((_BEB2,), jnp.int32), pltpu.VMEM((_BEB2,), jnp.int32),
            pltpu.VMEM((_BEB2, _HP), f32), pltpu.VMEM((_BEB2, _HP), f32),
            pltpu.VMEM((_BEB2, _HP), f32), pltpu.VMEM((_BEB2, _HP), f32),
            pltpu.VMEM((_BEB2,), jnp.int32), pltpu.VMEM((_BEB2,), jnp.int32),
            pltpu.VMEM((_BEB2, _HC), f32), pltpu.VMEM((_BEB2, _HC), f32),
            pltpu.VMEM((_BEB2, _HP), f32), pltpu.VMEM((_BEB2, _HP), f32),
            pltpu.SemaphoreType.DMA, pltpu.SemaphoreType.DMA,
            pltpu.SemaphoreType.DMA, pltpu.SemaphoreType.DMA,
            pltpu.VMEM_SHARED((_NPAD, _HP), f32),
        ],
    )(sd3, pb3, invd, h2, zeros)

    o0, o1 = pl.pallas_call(
        _tc_out_body,
        grid=(_N // _BN,),
        in_specs=[
            pl.BlockSpec((_NC, _BN, _HP), lambda i: (0, i, 0)),
            pl.BlockSpec((1, _HP), lambda i: (0, 0)),
            pl.BlockSpec((1, _HP), lambda i: (0, 0)),
        ],
        out_specs=[
            pl.BlockSpec((_BN, _HP), lambda i: (i, 0)),
            pl.BlockSpec((_BN, _HP), lambda i: (i, 0)),
        ],
        out_shape=[
            jax.ShapeDtypeStruct((_N, _HP), f32),
            jax.ShapeDtypeStruct((_N, _HP), f32),
        ],
    )(oh, bias[:_HP].reshape(1, _HP), bias[_HP:].reshape(1, _HP))
    return jnp.concatenate([o0, o1], axis=1)


# speedup vs baseline: 1.1213x; 1.1213x over previous
import jax
import jax.numpy as jnp
from jax import lax
from jax.experimental import pallas as pl
from jax.experimental.pallas import tpu as pltpu
from jax.experimental.pallas import tpu_sc as plsc

_N = 100000
_E = 1600000
_INDIM = 12
_OUTDIM = 32
_HEADS = 6
_EDGE_DIM_ = 4
_HP = 16
_HC = 96
_BE = 128
_NBLK = _E // _BE
_NC, _NS = 2, 16
_NW = _NC * _NS
_NPAD = 100096
_RPS = _NPAD // _NS
_BN = 2000
_BEB = 8000
_QA, _RA = _NBLK // _NW, _NBLK % _NW
_BEB2 = 64
_NBLKB = _E // _BEB2
_QB, _RB = _NBLKB // _NS, _NBLKB % _NS


def _group_mat():
    r = (lax.broadcasted_iota(jnp.int32, (_HEADS * _OUTDIM, _HP), 0) % _HC) // _HP
    c = lax.broadcasted_iota(jnp.int32, (_HEADS * _OUTDIM, _HP), 1)
    return (r == c).astype(jnp.float32)


def _tc_edge_body(ea_ref, bm_ref, pe_ref):
    pe_ref[...] = jnp.dot(ea_ref[...], bm_ref[...],
                          preferred_element_type=jnp.float32)


def _tc_node_body(x_ref, w_ref, asf_ref, adf_ref, h_ref, asrc_ref, adst_ref):
    h = jnp.dot(x_ref[...], w_ref[...], preferred_element_type=jnp.float32)
    h_ref[...] = h
    gm = _group_mat()
    asrc_ref[...] = jnp.dot(h * asf_ref[...], gm,
                            preferred_element_type=jnp.float32)
    adst_ref[...] = jnp.dot(h * adf_ref[...], gm,
                            preferred_element_type=jnp.float32)


def _sc_a_body(srcA, dstA, pe3, asrc, adst, zeros,
               p3, sd3, dpart,
               src0, src1, dst0, dst1,
               pe0, pe1, as0, as1, ad0, ad1, pv0, pv1,
               sl0, sl1, sg0, sg1, sw0, sw1, den_sh):
    c = lax.axis_index("c")
    s = lax.axis_index("s")
    wid = s * _NC + c
    r0 = s * _RPS
    pltpu.sync_copy(zeros.at[pl.ds(r0, _RPS)], den_sh.at[pl.ds(r0, _RPS)])
    plsc.subcore_barrier()

    base = wid * _QA + jnp.minimum(wid, _RA)
    nb = _QA + (wid < _RA).astype(jnp.int32)
    srcv = (src0, src1)
    dstv = (dst0, dst1)
    pev = (pe0, pe1)
    asv = (as0, as1)
    adv = (ad0, ad1)
    pv = (pv0, pv1)
    sl = (sl0, sl1)
    sg = (sg0, sg1)
    sw = (sw0, sw1)

    def fire_lin(b, q):
        pltpu.async_copy(srcA.at[b, 0], srcv[q], sl[q])
        pltpu.async_copy(dstA.at[b, 0], dstv[q], sl[q])
        pltpu.async_copy(pe3.at[b], pev[q], sl[q])

    def substep(i, par):
        b = base + i
        q = 1 - par

        @pl.when(i < nb)
        def _():
            pltpu.make_async_copy(srcA.at[b, 0], srcv[par], sl[par]).wait()
            pltpu.make_async_copy(dstA.at[b, 0], dstv[par], sl[par]).wait()
            pltpu.make_async_copy(pe3.at[b], pev[par], sl[par]).wait()
            pltpu.async_copy(asrc.at[srcv[par]], asv[par], sg[par])
            pltpu.async_copy(adst.at[dstv[par]], adv[par], sg[par])

        @pl.when((i >= 1) & (i - 1 < nb))
        def _():
            pltpu.make_async_copy(asrc.at[srcv[q]], asv[q], sg[q]).wait()
            pltpu.make_async_copy(adst.at[dstv[q]], adv[q], sg[q]).wait()

            @pl.when(i >= 3)
            def _():
                pltpu.make_async_copy(pv[q], p3.at[b], sw[q]).wait()

            a = asv[q][...] + adv[q][...] + pev[q][...]
            a = jnp.where(a >= 0.0, a, 0.2 * a)
            pv[q][...] = jnp.exp(a)
            pltpu.async_copy(pv[q], p3.at[b - 1], sw[q])
            pltpu.async_copy(srcv[q], sd3.at[b - 1, 0], sw[q])
            pltpu.async_copy(dstv[q], sd3.at[b - 1, 1], sw[q])
            pltpu.sync_copy(pv[q], den_sh.at[dstv[q]], add=True)

        @pl.when(i + 1 < nb)
        def _():
            @pl.when(i >= 1)
            def _():
                pltpu.make_async_copy(srcv[q], sd3.at[b, 0], sw[q]).wait()
                pltpu.make_async_copy(dstv[q], sd3.at[b, 1], sw[q]).wait()

            fire_lin(b + 1, q)

    fire_lin(base, 0)

    def pair(i2, carry):
        substep(2 * i2, 0)
        substep(2 * i2 + 1, 1)
        return carry

    lax.fori_loop(0, (nb + 2) // 2, pair, 0)
    for qq in (0, 1):
        pltpu.make_async_copy(pv[qq], p3.at[0], sw[qq]).wait()
        pltpu.make_async_copy(srcv[qq], sd3.at[0, 0], sw[qq]).wait()
        pltpu.make_async_copy(dstv[qq], sd3.at[0, 1], sw[qq]).wait()
    plsc.subcore_barrier()
    pltpu.sync_copy(den_sh.at[pl.ds(r0, _RPS)], dpart.at[c, pl.ds(r0, _RPS)])


def _tc_inv_body(d_ref, o_ref):
    o_ref[...] = 1.0 / (d_ref[0] + d_ref[1] + 1e-16)


def _sc_b_body(sd3, p3, invd, h2, zeros,
               oh,
               src0, src1, dst0, dst1, pb0, pb1, iv0, iv1,
               ix0, ix1, hr0, hr1, w_v, msg_v,
               sl0, sl1, sg0, sg1, acc_sh):
    c = lax.axis_index("c")
    s = lax.axis_index("s")
    r0 = s * _RPS
    pltpu.sync_copy(zeros.at[pl.ds(r0, _RPS)], acc_sh.at[pl.ds(r0, _RPS)])
    plsc.subcore_barrier()

    base = s * _QB + jnp.minimum(s, _RB)
    nb = _QB + (s < _RB).astype(jnp.int32)
    srcv = (src0, src1)
    dstv = (dst0, dst1)
    pbv = (pb0, pb1)
    iv = (iv0, iv1)
    ix = (ix0, ix1)
    hr = (hr0, hr1)
    sl = (sl0, sl1)
    sg = (sg0, sg1)

    def fire_lin(b, q):
        off = (b % 2) * _BEB2
        pltpu.async_copy(sd3.at[b // 2, 0, pl.ds(off, _BEB2)], srcv[q], sl[q])
        pltpu.async_copy(sd3.at[b // 2, 1, pl.ds(off, _BEB2)], dstv[q], sl[q])
        pltpu.async_copy(p3.at[b], pbv[q], sl[q])

    def substep(i, par):
        b = base + i
        q = 1 - par

        @pl.when(i < nb)
        def _():
            off = (b % 2) * _BEB2
            pltpu.make_async_copy(sd3.at[b // 2, 0, pl.ds(off, _BEB2)],
                                  srcv[par], sl[par]).wait()
            pltpu.make_async_copy(sd3.at[b // 2, 1, pl.ds(off, _BEB2)],
                                  dstv[par], sl[par]).wait()
            pltpu.make_async_copy(p3.at[b], pbv[par], sl[par]).wait()
            pltpu.async_copy(invd.at[dstv[par]], iv[par], sg[par])
            ix[par][...] = srcv[par][...] * 2 + c
            pltpu.async_copy(h2.at[ix[par]], hr[par], sg[par])

        @pl.when((i >= 1) & (i - 1 < nb))
        def _():
            pltpu.make_async_copy(invd.at[dstv[q]], iv[q], sg[q]).wait()
            pltpu.make_async_copy(h2.at[ix[q]], hr[q], sg[q]).wait()
            w_v[...] = pbv[q][...] * iv[q][...]
            hq = hr[q]

            def edge(j, carry2):
                w = w_v[j]
                m = w[0] * hq[j, pl.ds(0, _HP)]
                m = m + w[1] * hq[j, pl.ds(16, _HP)]
                m = m + w[2] * hq[j, pl.ds(32, _HP)]
                m = m + w[3] * hq[j, pl.ds(48, _HP)]
                m = m + w[4] * hq[j, pl.ds(64, _HP)]
                m = m + w[5] * hq[j, pl.ds(80, _HP)]
                msg_v[j] = m
                return carry2

            lax.fori_loop(0, _BEB2, edge, 0, unroll=4)
            pltpu.sync_copy(msg_v, acc_sh.at[dstv[q]], add=True)

        @pl.when(i + 1 < nb)
        def _():
            fire_lin(b + 1, q)

    fire_lin(base, 0)

    def pair(i2, carry):
        substep(2 * i2, 0)
        substep(2 * i2 + 1, 1)
        return carry

    lax.fori_loop(0, (nb + 2) // 2, pair, 0)
    plsc.subcore_barrier()
    pltpu.sync_copy(acc_sh.at[pl.ds(r0, _RPS)], oh.at[c, pl.ds(r0, _RPS)])


def _tc_out_body(oh_ref, b0_ref, b1_ref, o0_ref, o1_ref):
    o0_ref[...] = jax.nn.sigmoid(oh_ref[0] * (1.0 / _HEADS) + b0_ref[...])
    o1_ref[...] = jax.nn.sigmoid(oh_ref[1] * (1.0 / _HEADS) + b1_ref[...])


def kernel(x, edge_index, edge_attr, W, lin_edge, att_src, att_dst, att_edge,
           bias):
    f32 = jnp.float32
    srcA = edge_index[0].reshape(_NBLK, 1, _BE)
    dstA = edge_index[1].reshape(_NBLK, 1, _BE)
    Wp = W.reshape(_INDIM, _HEADS, 2, _HP).transpose(0, 2, 1, 3) \
          .reshape(_INDIM, _HEADS * _OUTDIM)
    asf = att_src.reshape(_HEADS, 2, _HP).transpose(1, 0, 2).reshape(1, -1)
    adf = att_dst.reshape(_HEADS, 2, _HP).transpose(1, 0, 2).reshape(1, -1)
    bmat = jnp.einsum('khd,hd->kh', lin_edge.reshape(_EDGE_DIM_, _HEADS,
                                                     _OUTDIM), att_edge)
    bmat = jnp.pad(bmat, ((0, 0), (0, _HP - _HEADS))).astype(f32)
    zeros = jnp.zeros((_NPAD, _HP), f32)

    hp, asrc, adst = pl.pallas_call(
        _tc_node_body,
        grid=(_N // _BN,),
        in_specs=[
            pl.BlockSpec((_BN, _INDIM), lambda i: (i, 0)),
            pl.BlockSpec((_INDIM, _HEADS * _OUTDIM), lambda i: (0, 0)),
            pl.BlockSpec((1, _HEADS * _OUTDIM), lambda i: (0, 0)),
            pl.BlockSpec((1, _HEADS * _OUTDIM), lambda i: (0, 0)),
        ],
        out_specs=[
            pl.BlockSpec((_BN, _HEADS * _OUTDIM), lambda i: (i, 0)),
            pl.BlockSpec((_BN, _HP), lambda i: (i, 0)),
            pl.BlockSpec((_BN, _HP), lambda i: (i, 0)),
        ],
        out_shape=[
            jax.ShapeDtypeStruct((_N, _HEADS * _OUTDIM), f32),
            jax.ShapeDtypeStruct((_N, _HP), f32),
            jax.ShapeDtypeStruct((_N, _HP), f32),
        ],
    )(x, Wp, asf, adf)

    pe = pl.pallas_call(
        _tc_edge_body,
        grid=(_E // _BEB,),
        in_specs=[
            pl.BlockSpec((_BEB, _EDGE_DIM_), lambda i: (i, 0)),
            pl.BlockSpec((_EDGE_DIM_, _HP), lambda i: (0, 0)),
        ],
        out_specs=pl.BlockSpec((_BEB, _HP), lambda i: (i, 0)),
        out_shape=jax.ShapeDtypeStruct((_E, _HP), f32),
    )(edge_attr, bmat)
    pe3 = pe.reshape(_NBLK, _BE, _HP)

    mesh = plsc.VectorSubcoreMesh(core_axis_name="c", subcore_axis_name="s",
                                  num_cores=_NC, num_subcores=_NS)
    p3, sd3, dpart = pl.kernel(
        _sc_a_body,
        out_type=[
            jax.ShapeDtypeStruct((_NBLK, _BE, _HP), f32),
            jax.ShapeDtypeStruct((_NBLK, 2, _BE), jnp.int32),
            jax.ShapeDtypeStruct((_NC, _NPAD, _HP), f32),
        ],
        mesh=mesh,
        compiler_params=pltpu.CompilerParams(use_tc_tiling_on_sc=False),
        scratch_types=[
            pltpu.VMEM((_BE,), jnp.int32), pltpu.VMEM((_BE,), jnp.int32),
            pltpu.VMEM((_BE,), jnp.int32), pltpu.VMEM((_BE,), jnp.int32),
            pltpu.VMEM((_BE, _HP), f32), pltpu.VMEM((_BE, _HP), f32),
            pltpu.VMEM((_BE, _HP), f32), pltpu.VMEM((_BE, _HP), f32),
            pltpu.VMEM((_BE, _HP), f32), pltpu.VMEM((_BE, _HP), f32),
            pltpu.VMEM((_BE, _HP), f32), pltpu.VMEM((_BE, _HP), f32),
            pltpu.SemaphoreType.DMA, pltpu.SemaphoreType.DMA,
            pltpu.SemaphoreType.DMA, pltpu.SemaphoreType.DMA,
            pltpu.SemaphoreType.DMA, pltpu.SemaphoreType.DMA,
            pltpu.VMEM_SHARED((_NPAD, _HP), f32),
        ],
    )(srcA, dstA, pe3, asrc, adst, zeros)

    invd = pl.pallas_call(
        _tc_inv_body,
        grid=(_NPAD // 3128,),
        in_specs=[pl.BlockSpec((_NC, 3128, _HP), lambda i: (0, i, 0))],
        out_specs=pl.BlockSpec((3128, _HP), lambda i: (i, 0)),
        out_shape=jax.ShapeDtypeStruct((_NPAD, _HP), f32),
    )(dpart)

    h2 = hp.reshape(_N * 2, _HC)
    pb3 = p3.reshape(_NBLKB, _BEB2, _HP)
    oh = pl.kernel(
        _sc_b_body,
        out_type=jax.ShapeDtypeStruct((_NC, _NPAD, _HP), f32),
        mesh=mesh,
        compiler_params=pltpu.CompilerParams(use_tc_tiling_on_sc=False),
        scratch_types=[
            pltpu.VMEM((_BEB2,), jnp.int32), pltpu.VMEM((_BEB2,), jnp.int32),
            pltpu.VMEM((_BEB2,), jnp.int32), pltpu.VMEM((_BEB2,), jnp.int32),
            pltpu.VMEM((_BEB2, _HP), f32), pltpu.VMEM((_BEB2, _HP), f32),
            pltpu.VMEM((_BEB2, _HP), f32), pltpu.VMEM((_BEB2, _HP), f32),
            pltpu.VMEM((_BEB2,), jnp.int32), pltpu.VMEM((_BEB2,), jnp.int32),
            pltpu.VMEM((_BEB2, _HC), f32), pltpu.VMEM((_BEB2, _HC), f32),
            pltpu.VMEM((_BEB2, _HP), f32), pltpu.VMEM((_BEB2, _HP), f32),
            pltpu.SemaphoreType.DMA, pltpu.SemaphoreType.DMA,
            pltpu.SemaphoreType.DMA, pltpu.SemaphoreType.DMA,
            pltpu.VMEM_SHARED((_NPAD, _HP), f32),
        ],
    )(sd3, pb3, invd, h2, zeros)

    o0, o1 = pl.pallas_call(
        _tc_out_body,
        grid=(_N // _BN,),
        in_specs=[
            pl.BlockSpec((_NC, _BN, _HP), lambda i: (0, i, 0)),
            pl.BlockSpec((1, _HP), lambda i: (0, 0)),
            pl.BlockSpec((1, _HP), lambda i: (0, 0)),
        ],
        out_specs=[
            pl.BlockSpec((_BN, _HP), lambda i: (i, 0)),
            pl.BlockSpec((_BN, _HP), lambda i: (i, 0)),
        ],
        out_shape=[
            jax.ShapeDtypeStruct((_N, _HP), f32),
            jax.ShapeDtypeStruct((_N, _HP), f32),
        ],
    )(oh, bias[:_HP].reshape(1, _HP), bias[_HP:].reshape(1, _HP))
    return jnp.concatenate([o0, o1], axis=1)
